# Initial kernel scaffold; baseline (speedup 1.0000x reference)
import jax
import jax.numpy as jnp
from jax.experimental import pallas as pl


def kernel(x, r_w1, r_b1, ln_scale, ln_bias, r_w2, r_b2, ew1, eb1, ew2, eb2, expert_priors):
    raise NotImplementedError("write your pallas kernel here")


# speedup vs baseline: 1.8159x; 1.8159x over previous
import functools

import jax
import jax.numpy as jnp
from jax.experimental import pallas as pl
from jax.experimental.pallas import tpu as pltpu

EMBED_DIM = 768
FF_DIM = 3072
NUM_EXPERTS = 8
TOP_K = 2

_SQRT_2_PI = 0.7978845608028654
_GELU_COEF = 0.044715


def _gelu(x):
    x3 = x * x * x
    inner = _SQRT_2_PI * (x + _GELU_COEF * x3)
    return 0.5 * x * (1.0 + jnp.tanh(inner))


def _router_kernel(x_ref, w1_ref, b1_ref, lns_ref, lnb_ref, w2_ref, b2_ref,
                   mask_ref):
    x = x_ref[...]
    h = jnp.dot(x, w1_ref[...], preferred_element_type=jnp.float32)
    h = h + b1_ref[...]
    mean = jnp.mean(h, axis=-1, keepdims=True)
    var = jnp.mean(jnp.square(h - mean), axis=-1, keepdims=True)
    h = (h - mean) * jax.lax.rsqrt(var + 1e-6) * lns_ref[...] + lnb_ref[...]
    h = _gelu(h)
    logits = jnp.dot(h, w2_ref[...], preferred_element_type=jnp.float32)
    logits = logits + b2_ref[...]
    lmax = jnp.max(logits, axis=-1, keepdims=True)
    ex = jnp.exp(logits - lmax)
    p = ex / jnp.sum(ex, axis=-1, keepdims=True)
    col = jax.lax.broadcasted_iota(jnp.int32, p.shape, 1)
    m1 = jnp.max(p, axis=-1, keepdims=True)
    idx1 = jnp.min(jnp.where(p == m1, col, NUM_EXPERTS), axis=-1, keepdims=True)
    sel1 = col == idx1
    p_wo = jnp.where(sel1, -jnp.inf, p)
    m2 = jnp.max(p_wo, axis=-1, keepdims=True)
    idx2 = jnp.min(jnp.where(p_wo == m2, col, NUM_EXPERTS), axis=-1,
                   keepdims=True)
    sel2 = col == idx2
    s = m1 + m2
    mask_ref[...] = (jnp.where(sel1, m1, 0.0) + jnp.where(sel2, m2, 0.0)) / s


def _expert_kernel(x_ref, mask_ref, w1a_ref, w1b_ref, w2_ref, b1a_ref,
                   b1b_ref, b2_ref, out_ref):
    e = pl.program_id(0)
    ft = pl.program_id(1)

    @pl.when((e == 0) & (ft == 0))
    def _init():
        out_ref[...] = jnp.zeros_like(out_ref)

    x = x_ref[...]
    h1 = jnp.dot(x, w1a_ref[0], preferred_element_type=jnp.float32)
    h1 = h1 + b1a_ref[0]
    h2 = jnp.dot(x, w1b_ref[0], preferred_element_type=jnp.float32)
    h2 = h2 + b1b_ref[0]
    g = h1 * _gelu(h2)
    contrib = jnp.dot(g, w2_ref[0], preferred_element_type=jnp.float32)

    col = jax.lax.broadcasted_iota(jnp.int32, mask_ref.shape, 1)
    mcol = jnp.sum(jnp.where(col == e, mask_ref[...], 0.0), axis=-1,
                   keepdims=True)
    out_ref[...] += contrib * mcol

    @pl.when(ft == 0)
    def _bias2():
        out_ref[...] += mcol * b2_ref[0]


def kernel(x, r_w1, r_b1, ln_scale, ln_bias, r_w2, r_b2, ew1, eb1, ew2, eb2,
           expert_priors):
    B, S, D = x.shape
    E = r_b2.shape[0]
    F = FF_DIM
    x2d = x.reshape(S, D)

    mask = pl.pallas_call(
        _router_kernel,
        out_shape=jax.ShapeDtypeStruct((S, E), jnp.float32),
        in_specs=[pl.BlockSpec((S, D), lambda: (0, 0)),
                  pl.BlockSpec((D, D // 2), lambda: (0, 0)),
                  pl.BlockSpec((1, D // 2), lambda: (0, 0)),
                  pl.BlockSpec((1, D // 2), lambda: (0, 0)),
                  pl.BlockSpec((1, D // 2), lambda: (0, 0)),
                  pl.BlockSpec((D // 2, E), lambda: (0, 0)),
                  pl.BlockSpec((1, E), lambda: (0, 0))],
        out_specs=pl.BlockSpec((S, E), lambda: (0, 0)),
    )(x2d, r_w1, r_b1.reshape(1, -1), ln_scale.reshape(1, -1),
      ln_bias.reshape(1, -1), r_w2, r_b2.reshape(1, -1))

    FT = 512
    n_ft = F // FT
    n2ft = 2 * F // FT
    eb1r = eb1.reshape(E * n2ft, 1, FT)
    eb2r = eb2.reshape(E, 1, D)

    out = pl.pallas_call(
        _expert_kernel,
        grid=(E, n_ft),
        out_shape=jax.ShapeDtypeStruct((S, D), jnp.float32),
        in_specs=[
            pl.BlockSpec((S, D), lambda e, f: (0, 0)),
            pl.BlockSpec((S, E), lambda e, f: (0, 0)),
            pl.BlockSpec((1, D, FT), lambda e, f: (e, 0, f)),
            pl.BlockSpec((1, D, FT), lambda e, f: (e, 0, f + n_ft)),
            pl.BlockSpec((1, FT, D), lambda e, f: (e, f, 0)),
            pl.BlockSpec((1, 1, FT), lambda e, f: (e * n2ft + f, 0, 0)),
            pl.BlockSpec((1, 1, FT), lambda e, f: (e * n2ft + f + n_ft, 0, 0)),
            pl.BlockSpec((1, 1, D), lambda e, f: (e, 0, 0)),
        ],
        out_specs=pl.BlockSpec((S, D), lambda e, f: (0, 0)),
        compiler_params=pltpu.CompilerParams(
            dimension_semantics=("arbitrary", "arbitrary")),
    )(x2d, mask, ew1, ew1, ew2, eb1r, eb1r, eb2r)

    return (out.reshape(B, S, D), 0.0)
